# baseline (speedup 1.0000x reference)
import jax
import jax.numpy as jnp
from jax import lax
from jax.experimental import pallas as pl
from jax.experimental.pallas import tpu as pltpu, tpu_sc as plsc

NC, NS, L = 2, 16, 16
NW = NC * NS

B = 16384
D = 128
BPW = B // NW
C = 64
NB = 4
NCHUNK = BPW // C
G = C // L


def _mf_body(users, items, ut, it, out, uidx, iidx, urows, irows, tpose,
             outbuf, sem_u, sem_i, sem_idx, sem_out):
    wid = lax.axis_index("s") * NC + lax.axis_index("c")
    base = wid * BPW
    cpu = pltpu.async_copy(users.at[pl.ds(base, BPW)], uidx, sem_idx.at[0])
    cpi = pltpu.async_copy(items.at[pl.ds(base, BPW)], iidx, sem_idx.at[1])
    cpu.wait()
    cpi.wait()
    lane = lax.iota(jnp.int32, 16)

    def start(c, b):
        pltpu.async_copy(ut.at[uidx.at[pl.ds(c * C, C)]], urows.at[b],
                         sem_u.at[b])
        pltpu.async_copy(it.at[iidx.at[pl.ds(c * C, C)]], irows.at[b],
                         sem_i.at[b])

    for p in range(NB):
        start(p, p)

    @pl.loop(0, NCHUNK)
    def _chunk(c):
        b = lax.rem(c, NB)
        with jax.named_scope("dma_wait"):
            pltpu.make_async_copy(ut.at[uidx.at[pl.ds(c * C, C)]], urows.at[b],
                                  sem_u.at[b]).wait()
            pltpu.make_async_copy(it.at[iidx.at[pl.ds(c * C, C)]], irows.at[b],
                                  sem_i.at[b]).wait()

        with jax.named_scope("compute"):
            @pl.loop(0, G)
            def _grp(g):
                for rr in range(L):
                    r = g * L + rr
                    accs = [urows[b, r, pl.ds(k * L, L)]
                            * irows[b, r, pl.ds(k * L, L)] for k in range(4)]
                    for k in range(4, D // L):
                        accs[k % 4] = accs[k % 4] + (
                            urows[b, r, pl.ds(k * L, L)]
                            * irows[b, r, pl.ds(k * L, L)])
                    tpose[rr, :] = (accs[0] + accs[1]) + (accs[2] + accs[3])
                res0 = plsc.load_gather(tpose, [lane, lane])
                res1 = plsc.load_gather(tpose, [lane, (lane + 1) & (L - 1)])
                for k in range(2, L, 2):
                    res0 = res0 + plsc.load_gather(tpose, [lane, (lane + k) & (L - 1)])
                    res1 = res1 + plsc.load_gather(tpose, [lane, (lane + k + 1) & (L - 1)])
                outbuf[pl.ds(c * C + g * L, L)] = (res0 + res1) * (1.0 / D)

        pltpu.async_copy(outbuf.at[pl.ds(c * C, C)],
                         out.at[pl.ds(base + c * C, C)], sem_out)

        @pl.when(c + NB < NCHUNK)
        def _():
            start(c + NB, b)

    pltpu.make_async_copy(outbuf, out.at[pl.ds(base, BPW)], sem_out).wait()


@jax.jit
def kernel(users, items, user_table, item_table):
    mesh = plsc.VectorSubcoreMesh(core_axis_name="c", subcore_axis_name="s")
    f = pl.kernel(
        _mf_body,
        out_type=jax.ShapeDtypeStruct((B,), jnp.float32),
        mesh=mesh,
        compiler_params=pltpu.CompilerParams(needs_layout_passes=False),
        scratch_types=[
            pltpu.VMEM((BPW,), jnp.int32),
            pltpu.VMEM((BPW,), jnp.int32),
            pltpu.VMEM((NB, C, D), jnp.float32),
            pltpu.VMEM((NB, C, D), jnp.float32),
            pltpu.VMEM((L, L), jnp.float32),
            pltpu.VMEM((BPW,), jnp.float32),
            pltpu.SemaphoreType.DMA((NB,)),
            pltpu.SemaphoreType.DMA((NB,)),
            pltpu.SemaphoreType.DMA((2,)),
            pltpu.SemaphoreType.DMA,
        ],
    )
    return f(users.astype(jnp.int32), items.astype(jnp.int32),
             user_table, item_table)


# speedup vs baseline: 1.0210x; 1.0210x over previous
import jax
import jax.numpy as jnp
from jax import lax
from jax.experimental import pallas as pl
from jax.experimental.pallas import tpu as pltpu, tpu_sc as plsc

NC, NS, L = 2, 16, 16
NW = NC * NS

B = 16384
D = 128
BPW = B // NW
C = 64
NB = 4
NCHUNK = BPW // C
G = C // L


def _mf_body(users, items, ut, it, out, uidx, iidx, urows, irows, tpose,
             outbuf, sem_u, sem_i, sem_idx, sem_out):
    wid = lax.axis_index("s") * NC + lax.axis_index("c")
    base = wid * BPW
    cpu = pltpu.async_copy(users.at[pl.ds(base, BPW)], uidx, sem_idx.at[0])
    cpi = pltpu.async_copy(items.at[pl.ds(base, BPW)], iidx, sem_idx.at[1])
    cpu.wait()
    cpi.wait()
    lane = lax.iota(jnp.int32, 16)

    def start(c, b):
        pltpu.async_copy(ut.at[uidx.at[pl.ds(c * C, C)]], urows.at[b],
                         sem_u.at[b])
        pltpu.async_copy(it.at[iidx.at[pl.ds(c * C, C)]], irows.at[b],
                         sem_i.at[b])

    for p in range(NB):
        start(p, p)

    @pl.loop(0, NCHUNK)
    def _chunk(c):
        b = lax.rem(c, NB)
        with jax.named_scope("dma_wait"):
            pltpu.make_async_copy(ut.at[uidx.at[pl.ds(c * C, C)]], urows.at[b],
                                  sem_u.at[b]).wait()
            pltpu.make_async_copy(it.at[iidx.at[pl.ds(c * C, C)]], irows.at[b],
                                  sem_i.at[b]).wait()

        with jax.named_scope("compute"):
            @pl.loop(0, G)
            def _grp(g):
                for rr in range(L):
                    r = g * L + rr
                    acc0 = urows[b, r, pl.ds(0, L)] * irows[b, r, pl.ds(0, L)]
                    acc1 = urows[b, r, pl.ds(L, L)] * irows[b, r, pl.ds(L, L)]
                    for k in range(2, D // L, 2):
                        acc0 = acc0 + (urows[b, r, pl.ds(k * L, L)]
                                       * irows[b, r, pl.ds(k * L, L)])
                        acc1 = acc1 + (urows[b, r, pl.ds((k + 1) * L, L)]
                                       * irows[b, r, pl.ds((k + 1) * L, L)])
                    tpose[rr, :] = acc0 + acc1
                res0 = plsc.load_gather(tpose, [lane, lane])
                res1 = plsc.load_gather(tpose, [lane, (lane + 1) & (L - 1)])
                for k in range(2, L, 2):
                    res0 = res0 + plsc.load_gather(tpose, [lane, (lane + k) & (L - 1)])
                    res1 = res1 + plsc.load_gather(tpose, [lane, (lane + k + 1) & (L - 1)])
                outbuf[pl.ds(c * C + g * L, L)] = (res0 + res1) * (1.0 / D)

        pltpu.async_copy(outbuf.at[pl.ds(c * C, C)],
                         out.at[pl.ds(base + c * C, C)], sem_out)

        @pl.when(c + NB < NCHUNK)
        def _():
            start(c + NB, b)

    pltpu.make_async_copy(outbuf, out.at[pl.ds(base, BPW)], sem_out).wait()


@jax.jit
def kernel(users, items, user_table, item_table):
    mesh = plsc.VectorSubcoreMesh(core_axis_name="c", subcore_axis_name="s")
    f = pl.kernel(
        _mf_body,
        out_type=jax.ShapeDtypeStruct((B,), jnp.float32),
        mesh=mesh,
        compiler_params=pltpu.CompilerParams(needs_layout_passes=False),
        scratch_types=[
            pltpu.VMEM((BPW,), jnp.int32),
            pltpu.VMEM((BPW,), jnp.int32),
            pltpu.VMEM((NB, C, D), jnp.float32),
            pltpu.VMEM((NB, C, D), jnp.float32),
            pltpu.VMEM((L, L), jnp.float32),
            pltpu.VMEM((BPW,), jnp.float32),
            pltpu.SemaphoreType.DMA((NB,)),
            pltpu.SemaphoreType.DMA((NB,)),
            pltpu.SemaphoreType.DMA((2,)),
            pltpu.SemaphoreType.DMA,
        ],
    )
    return f(users.astype(jnp.int32), items.astype(jnp.int32),
             user_table, item_table)


# speedup vs baseline: 1.0493x; 1.0277x over previous
import jax
import jax.numpy as jnp
from jax import lax
from jax.experimental import pallas as pl
from jax.experimental.pallas import tpu as pltpu, tpu_sc as plsc

NC, NS, L = 2, 16, 16
NW = NC * NS

B = 16384
D = 128
BPW = B // NW
C = 32
NB = 8
NCHUNK = BPW // C
G = C // L


def _mf_body(users, items, ut, it, out, uidx, iidx, urows, irows, tpose,
             outbuf, sem_u, sem_i, sem_idx, sem_out):
    wid = lax.axis_index("s") * NC + lax.axis_index("c")
    base = wid * BPW
    cpu = pltpu.async_copy(users.at[pl.ds(base, BPW)], uidx, sem_idx.at[0])
    cpi = pltpu.async_copy(items.at[pl.ds(base, BPW)], iidx, sem_idx.at[1])
    cpu.wait()
    cpi.wait()
    lane = lax.iota(jnp.int32, 16)

    def start(c, b):
        pltpu.async_copy(ut.at[uidx.at[pl.ds(c * C, C)]], urows.at[b],
                         sem_u.at[b])
        pltpu.async_copy(it.at[iidx.at[pl.ds(c * C, C)]], irows.at[b],
                         sem_i.at[b])

    for p in range(NB):
        start(p, p)

    @pl.loop(0, NCHUNK)
    def _chunk(c):
        b = lax.rem(c, NB)
        with jax.named_scope("dma_wait"):
            pltpu.make_async_copy(ut.at[uidx.at[pl.ds(c * C, C)]], urows.at[b],
                                  sem_u.at[b]).wait()
            pltpu.make_async_copy(it.at[iidx.at[pl.ds(c * C, C)]], irows.at[b],
                                  sem_i.at[b]).wait()

        with jax.named_scope("compute"):
            @pl.loop(0, G)
            def _grp(g):
                for rr in range(L):
                    r = g * L + rr
                    acc0 = urows[b, r, pl.ds(0, L)] * irows[b, r, pl.ds(0, L)]
                    acc1 = urows[b, r, pl.ds(L, L)] * irows[b, r, pl.ds(L, L)]
                    for k in range(2, D // L, 2):
                        acc0 = acc0 + (urows[b, r, pl.ds(k * L, L)]
                                       * irows[b, r, pl.ds(k * L, L)])
                        acc1 = acc1 + (urows[b, r, pl.ds((k + 1) * L, L)]
                                       * irows[b, r, pl.ds((k + 1) * L, L)])
                    tpose[rr, :] = acc0 + acc1
                res0 = plsc.load_gather(tpose, [lane, lane])
                res1 = plsc.load_gather(tpose, [lane, (lane + 1) & (L - 1)])
                for k in range(2, L, 2):
                    res0 = res0 + plsc.load_gather(tpose, [lane, (lane + k) & (L - 1)])
                    res1 = res1 + plsc.load_gather(tpose, [lane, (lane + k + 1) & (L - 1)])
                outbuf[pl.ds(c * C + g * L, L)] = (res0 + res1) * (1.0 / D)

        pltpu.async_copy(outbuf.at[pl.ds(c * C, C)],
                         out.at[pl.ds(base + c * C, C)], sem_out)

        @pl.when(c + NB < NCHUNK)
        def _():
            start(c + NB, b)

    pltpu.make_async_copy(outbuf, out.at[pl.ds(base, BPW)], sem_out).wait()


@jax.jit
def kernel(users, items, user_table, item_table):
    mesh = plsc.VectorSubcoreMesh(core_axis_name="c", subcore_axis_name="s")
    f = pl.kernel(
        _mf_body,
        out_type=jax.ShapeDtypeStruct((B,), jnp.float32),
        mesh=mesh,
        compiler_params=pltpu.CompilerParams(needs_layout_passes=False),
        scratch_types=[
            pltpu.VMEM((BPW,), jnp.int32),
            pltpu.VMEM((BPW,), jnp.int32),
            pltpu.VMEM((NB, C, D), jnp.float32),
            pltpu.VMEM((NB, C, D), jnp.float32),
            pltpu.VMEM((L, L), jnp.float32),
            pltpu.VMEM((BPW,), jnp.float32),
            pltpu.SemaphoreType.DMA((NB,)),
            pltpu.SemaphoreType.DMA((NB,)),
            pltpu.SemaphoreType.DMA((2,)),
            pltpu.SemaphoreType.DMA,
        ],
    )
    return f(users.astype(jnp.int32), items.astype(jnp.int32),
             user_table, item_table)


# speedup vs baseline: 1.0694x; 1.0192x over previous
import jax
import jax.numpy as jnp
from jax import lax
from jax.experimental import pallas as pl
from jax.experimental.pallas import tpu as pltpu, tpu_sc as plsc

NC, NS, L = 2, 16, 16
NW = NC * NS

B = 16384
D = 128
BPW = B // NW
C = 16
NB = 8
NCHUNK = BPW // C
G = C // L


def _mf_body(users, items, ut, it, out, uidx, iidx, urows, irows, tpose,
             outbuf, sem_u, sem_i, sem_idx, sem_out):
    wid = lax.axis_index("s") * NC + lax.axis_index("c")
    base = wid * BPW
    cpu = pltpu.async_copy(users.at[pl.ds(base, BPW)], uidx, sem_idx.at[0])
    cpi = pltpu.async_copy(items.at[pl.ds(base, BPW)], iidx, sem_idx.at[1])
    cpu.wait()
    cpi.wait()
    lane = lax.iota(jnp.int32, 16)

    def start(c, b):
        pltpu.async_copy(ut.at[uidx.at[pl.ds(c * C, C)]], urows.at[b],
                         sem_u.at[b])
        pltpu.async_copy(it.at[iidx.at[pl.ds(c * C, C)]], irows.at[b],
                         sem_i.at[b])

    for p in range(NB):
        start(p, p)

    @pl.loop(0, NCHUNK)
    def _chunk(c):
        b = lax.rem(c, NB)
        with jax.named_scope("dma_wait"):
            pltpu.make_async_copy(ut.at[uidx.at[pl.ds(c * C, C)]], urows.at[b],
                                  sem_u.at[b]).wait()
            pltpu.make_async_copy(it.at[iidx.at[pl.ds(c * C, C)]], irows.at[b],
                                  sem_i.at[b]).wait()

        with jax.named_scope("compute"):
            @pl.loop(0, G)
            def _grp(g):
                for rr in range(L):
                    r = g * L + rr
                    acc0 = urows[b, r, pl.ds(0, L)] * irows[b, r, pl.ds(0, L)]
                    acc1 = urows[b, r, pl.ds(L, L)] * irows[b, r, pl.ds(L, L)]
                    for k in range(2, D // L, 2):
                        acc0 = acc0 + (urows[b, r, pl.ds(k * L, L)]
                                       * irows[b, r, pl.ds(k * L, L)])
                        acc1 = acc1 + (urows[b, r, pl.ds((k + 1) * L, L)]
                                       * irows[b, r, pl.ds((k + 1) * L, L)])
                    tpose[rr, :] = acc0 + acc1
                res0 = plsc.load_gather(tpose, [lane, lane])
                res1 = plsc.load_gather(tpose, [lane, (lane + 1) & (L - 1)])
                for k in range(2, L, 2):
                    res0 = res0 + plsc.load_gather(tpose, [lane, (lane + k) & (L - 1)])
                    res1 = res1 + plsc.load_gather(tpose, [lane, (lane + k + 1) & (L - 1)])
                outbuf[pl.ds(c * C + g * L, L)] = (res0 + res1) * (1.0 / D)

        pltpu.async_copy(outbuf.at[pl.ds(c * C, C)],
                         out.at[pl.ds(base + c * C, C)], sem_out)

        @pl.when(c + NB < NCHUNK)
        def _():
            start(c + NB, b)

    pltpu.make_async_copy(outbuf, out.at[pl.ds(base, BPW)], sem_out).wait()


@jax.jit
def kernel(users, items, user_table, item_table):
    mesh = plsc.VectorSubcoreMesh(core_axis_name="c", subcore_axis_name="s")
    f = pl.kernel(
        _mf_body,
        out_type=jax.ShapeDtypeStruct((B,), jnp.float32),
        mesh=mesh,
        compiler_params=pltpu.CompilerParams(needs_layout_passes=False),
        scratch_types=[
            pltpu.VMEM((BPW,), jnp.int32),
            pltpu.VMEM((BPW,), jnp.int32),
            pltpu.VMEM((NB, C, D), jnp.float32),
            pltpu.VMEM((NB, C, D), jnp.float32),
            pltpu.VMEM((L, L), jnp.float32),
            pltpu.VMEM((BPW,), jnp.float32),
            pltpu.SemaphoreType.DMA((NB,)),
            pltpu.SemaphoreType.DMA((NB,)),
            pltpu.SemaphoreType.DMA((2,)),
            pltpu.SemaphoreType.DMA,
        ],
    )
    return f(users.astype(jnp.int32), items.astype(jnp.int32),
             user_table, item_table)


# speedup vs baseline: 1.0696x; 1.0002x over previous
import jax
import jax.numpy as jnp
from jax import lax
from jax.experimental import pallas as pl
from jax.experimental.pallas import tpu as pltpu, tpu_sc as plsc

NC, NS, L = 2, 16, 16
NW = NC * NS

B = 16384
D = 128
BPW = B // NW
C = 16
NB = 8
NCHUNK = BPW // C
G = C // L


def _mf_body(users, items, ut, it, out, uidx, iidx, urows, irows, tpose,
             outbuf, sem_u, sem_i, sem_idx, sem_out):
    wid = lax.axis_index("s") * NC + lax.axis_index("c")
    base = wid * BPW
    cpu = pltpu.async_copy(users.at[pl.ds(base, BPW)], uidx, sem_idx.at[0])
    cpi = pltpu.async_copy(items.at[pl.ds(base, BPW)], iidx, sem_idx.at[1])
    cpu.wait()
    cpi.wait()
    lane = lax.iota(jnp.int32, 16)

    def start(c, b):
        pltpu.async_copy(ut.at[uidx.at[pl.ds(c * C, C)]], urows.at[b],
                         sem_u.at[b])
        pltpu.async_copy(it.at[iidx.at[pl.ds(c * C, C)]], irows.at[b],
                         sem_i.at[b])

    for p in range(NB):
        start(p, p)

    @pl.loop(0, NCHUNK)
    def _chunk(c):
        b = lax.rem(c, NB)
        pltpu.make_async_copy(ut.at[uidx.at[pl.ds(c * C, C)]], urows.at[b],
                              sem_u.at[b]).wait()
        pltpu.make_async_copy(it.at[iidx.at[pl.ds(c * C, C)]], irows.at[b],
                              sem_i.at[b]).wait()

        @pl.loop(0, G)
        def _grp(g):
            for rr in range(L):
                r = g * L + rr
                acc0 = urows[b, r, pl.ds(0, L)] * irows[b, r, pl.ds(0, L)]
                acc1 = urows[b, r, pl.ds(L, L)] * irows[b, r, pl.ds(L, L)]
                for k in range(2, D // L, 2):
                    acc0 = acc0 + (urows[b, r, pl.ds(k * L, L)]
                                   * irows[b, r, pl.ds(k * L, L)])
                    acc1 = acc1 + (urows[b, r, pl.ds((k + 1) * L, L)]
                                   * irows[b, r, pl.ds((k + 1) * L, L)])
                tpose[rr, :] = acc0 + acc1
            res0 = plsc.load_gather(tpose, [lane, lane])
            res1 = plsc.load_gather(tpose, [lane, (lane + 1) & (L - 1)])
            for k in range(2, L, 2):
                res0 = res0 + plsc.load_gather(tpose, [lane, (lane + k) & (L - 1)])
                res1 = res1 + plsc.load_gather(tpose, [lane, (lane + k + 1) & (L - 1)])
            outbuf[pl.ds(c * C + g * L, L)] = (res0 + res1) * (1.0 / D)

        pltpu.async_copy(outbuf.at[pl.ds(c * C, C)],
                         out.at[pl.ds(base + c * C, C)], sem_out)

        @pl.when(c + NB < NCHUNK)
        def _():
            start(c + NB, b)

    pltpu.make_async_copy(outbuf, out.at[pl.ds(base, BPW)], sem_out).wait()


@jax.jit
def kernel(users, items, user_table, item_table):
    mesh = plsc.VectorSubcoreMesh(core_axis_name="c", subcore_axis_name="s")
    f = pl.kernel(
        _mf_body,
        out_type=jax.ShapeDtypeStruct((B,), jnp.float32),
        mesh=mesh,
        compiler_params=pltpu.CompilerParams(needs_layout_passes=False),
        scratch_types=[
            pltpu.VMEM((BPW,), jnp.int32),
            pltpu.VMEM((BPW,), jnp.int32),
            pltpu.VMEM((NB, C, D), jnp.float32),
            pltpu.VMEM((NB, C, D), jnp.float32),
            pltpu.VMEM((L, L), jnp.float32),
            pltpu.VMEM((BPW,), jnp.float32),
            pltpu.SemaphoreType.DMA((NB,)),
            pltpu.SemaphoreType.DMA((NB,)),
            pltpu.SemaphoreType.DMA((2,)),
            pltpu.SemaphoreType.DMA,
        ],
    )
    return f(users.astype(jnp.int32), items.astype(jnp.int32),
             user_table, item_table)
